# baseline (speedup 1.0000x reference)
import functools

import jax
import jax.numpy as jnp
from jax.experimental import pallas as pl
from jax.experimental.pallas import tpu as pltpu

_TINY = 1.1754943508222875e-38
_BIG_I32 = 2**31 - 1
_KS = (0, 42, 0x1BD11BDA ^ 0 ^ 42)
_ROTS = ((13, 15, 26, 6), (17, 29, 16, 24))


def _rotl(x, r):
    return (x << jnp.uint32(r)) | (x >> jnp.uint32(32 - r))


def _threefry_bits(f_u32):
    x1 = f_u32 + jnp.uint32(_KS[1])
    x0 = x1
    x1 = _rotl(x1, _ROTS[0][0]) ^ x0
    for i in range(5):
        rots = _ROTS[i % 2]
        for r in (rots[1:] if i == 0 else rots):
            x0 = x0 + x1
            x1 = _rotl(x1, r) ^ x0
        a = _KS[(i + 1) % 3]
        b = (_KS[(i + 2) % 3] + i + 1) & 0xFFFFFFFF
        if a:
            x0 = x0 + jnp.uint32(a)
        x1 = x1 + jnp.uint32(b)
    return x0 ^ x1


def _gumbel_from_bits(bits):
    fb = (bits >> jnp.uint32(9)) | jnp.uint32(0x3F800000)
    u01 = jax.lax.bitcast_convert_type(fb, jnp.float32) - jnp.float32(1.0)
    tiny = jnp.float32(_TINY)
    u = jnp.maximum(tiny, u01 + tiny)
    return -jnp.log(-jnp.log(u))


def _gumbel_at(f_i32):
    return _gumbel_from_bits(
        _threefry_bits(jax.lax.bitcast_convert_type(f_i32, jnp.uint32)))


def _reduce_rowwise(av, af):
    m = jnp.max(av, axis=1, keepdims=True)
    f_win = jnp.min(jnp.where(av == m, af, jnp.int32(_BIG_I32)),
                    axis=1, keepdims=True)
    return m, f_win


def _sample_kernel(V, BR, BEAM, VC, NC, NBUF,
                   lp_hbm, tail_ref, sc_ref, idx_ref, score_ref, buf, sem):
    i = pl.program_id(0)
    R = BR * BEAM
    TW = V - NC * VC

    def start_copy(j, slot):
        pltpu.make_async_copy(
            lp_hbm.at[pl.ds(i * BR, BR), :, pl.ds(j * VC, VC)],
            buf.at[slot], sem.at[slot]).start()

    def wait_copy(j, slot):
        pltpu.make_async_copy(
            lp_hbm.at[pl.ds(i * BR, BR), :, pl.ds(j * VC, VC)],
            buf.at[slot], sem.at[slot]).wait()

    for s in range(NBUF - 1):
        start_copy(s, s)

    lane = jax.lax.broadcasted_iota(jnp.int32, (R, VC), 1)
    row_v = (jax.lax.broadcasted_iota(jnp.int32, (R, VC), 0) + i * R) * V
    f_base = row_v + lane

    def chunk_val(j, slot):
        wait_copy(j, slot)
        lp = buf[slot].reshape(R, VC)
        g = _gumbel_at(f_base + j * VC)
        return g + lp, lp

    def body(j, carry):
        acc_v, acc_j = carry
        slot = jax.lax.rem(j, NBUF)
        nxt = j + (NBUF - 1)

        @pl.when(nxt < NC)
        def _():
            start_copy(nxt, jax.lax.rem(nxt, NBUF))

        val, _lp = chunk_val(j, slot)
        better = val > acc_v
        return (jnp.where(better, val, acc_v),
                jnp.where(better, jnp.int32(1) * (j * VC), acc_j))

    init = (jnp.full((R, VC), -jnp.inf, jnp.float32),
            jnp.zeros((R, VC), jnp.int32))
    acc_v, acc_j = jax.lax.fori_loop(0, NC, body, init, unroll=4)
    m1, f1 = _reduce_rowwise(acc_v, f_base + acc_j)

    if TW > 0:
        lane_t = jax.lax.broadcasted_iota(jnp.int32, (R, TW), 1)
        row_v_t = (jax.lax.broadcasted_iota(jnp.int32, (R, TW), 0) + i * R) * V
        f_t = row_v_t + lane_t + NC * VC
        lp_t = tail_ref[...].reshape(R, TW)
        val_t = _gumbel_at(f_t) + lp_t
        m2, f2 = _reduce_rowwise(val_t, f_t)
        tb = m2 > m1
        m1 = jnp.where(tb, m2, m1)
        f1 = jnp.where(tb, f2, f1)

    lp_win = m1 - _gumbel_at(f1)
    idx = f1 - (jax.lax.broadcasted_iota(jnp.int32, (R, 1), 0) + i * R) * V
    idx_ref[...] = idx.reshape(1, 1, R)
    score_ref[...] = lp_win.reshape(1, 1, R) + sc_ref[...]


def kernel(step, lprobs, scores):
    bsz, beam_size, V = lprobs.shape
    NROWS = bsz * beam_size
    BR = 4
    R = BR * beam_size
    VC = 512
    NC = V // VC
    NBUF = 4
    ni = NROWS // R

    sc = scores.reshape(NROWS).reshape(ni, 1, R).astype(jnp.float32)
    TW = V - NC * VC
    tail = jax.lax.slice_in_dim(lprobs, NC * VC, V, axis=2)

    idx3, score3 = pl.pallas_call(
        functools.partial(_sample_kernel, V, BR, beam_size, VC, NC, NBUF),
        grid=(ni,),
        in_specs=[
            pl.BlockSpec(memory_space=pltpu.MemorySpace.HBM),
            pl.BlockSpec((BR, beam_size, TW), lambda i: (i, 0, 0)),
            pl.BlockSpec((1, 1, R), lambda i: (i, 0, 0)),
        ],
        out_specs=[
            pl.BlockSpec((1, 1, R), lambda i: (i, 0, 0)),
            pl.BlockSpec((1, 1, R), lambda i: (i, 0, 0)),
        ],
        out_shape=[
            jax.ShapeDtypeStruct((ni, 1, R), jnp.int32),
            jax.ShapeDtypeStruct((ni, 1, R), jnp.float32),
        ],
        scratch_shapes=[
            pltpu.VMEM((4, BR, beam_size, VC), jnp.float32),
            pltpu.SemaphoreType.DMA((4,)),
        ],
        compiler_params=pltpu.CompilerParams(
            dimension_semantics=("arbitrary",),
        ),
    )(lprobs, tail, sc)

    indices_buf = idx3.reshape(bsz, beam_size)
    scores_buf = score3.reshape(bsz, beam_size)
    beams_buf = jnp.tile(jnp.arange(beam_size, dtype=indices_buf.dtype), (bsz, 1))
    return (scores_buf, indices_buf, beams_buf)


# speedup vs baseline: 1.6400x; 1.6400x over previous
import functools

import jax
import jax.numpy as jnp
from jax.experimental import pallas as pl
from jax.experimental.pallas import tpu as pltpu

_TINY = 1.1754943508222875e-38
_BIG_I32 = 2**31 - 1
_KS = (0, 42, 0x1BD11BDA ^ 0 ^ 42)
_ROTS = ((13, 15, 26, 6), (17, 29, 16, 24))


def _rotl(x, r):
    return (x << jnp.uint32(r)) | (x >> jnp.uint32(32 - r))


def _threefry_bits(f_u32):
    x1 = f_u32 + jnp.uint32(_KS[1])
    x0 = x1
    x1 = _rotl(x1, _ROTS[0][0]) ^ x0
    for i in range(5):
        rots = _ROTS[i % 2]
        for r in (rots[1:] if i == 0 else rots):
            x0 = x0 + x1
            x1 = _rotl(x1, r) ^ x0
        a = _KS[(i + 1) % 3]
        b = (_KS[(i + 2) % 3] + i + 1) & 0xFFFFFFFF
        if a:
            x0 = x0 + jnp.uint32(a)
        x1 = x1 + jnp.uint32(b)
    return x0 ^ x1


def _gumbel_from_bits(bits):
    fb = (bits >> jnp.uint32(9)) | jnp.uint32(0x3F800000)
    u01 = jax.lax.bitcast_convert_type(fb, jnp.float32) - jnp.float32(1.0)
    tiny = jnp.float32(_TINY)
    u = jnp.maximum(tiny, u01 + tiny)
    return -jnp.log(-jnp.log(u))


def _gumbel_at(f_i32):
    return _gumbel_from_bits(
        _threefry_bits(jax.lax.bitcast_convert_type(f_i32, jnp.uint32)))


def _reduce_rowwise(av, af):
    m = jnp.max(av, axis=1, keepdims=True)
    f_win = jnp.min(jnp.where(av == m, af, jnp.int32(_BIG_I32)),
                    axis=1, keepdims=True)
    return m, f_win


def _sample_kernel(V, BR, BEAM, VC, NC,
                   lp_hbm, sc_ref, idx_ref, score_ref, buf, sem):
    i = pl.program_id(0)
    ni = pl.num_programs(0)
    R = BR * BEAM
    TW = V - NC * VC

    def block_copy(b, slot):
        return pltpu.make_async_copy(
            lp_hbm.at[pl.ds(b * BR, BR)], buf.at[slot], sem.at[slot])

    slot = jax.lax.rem(i, 2)

    @pl.when(i == 0)
    def _prologue():
        block_copy(0, 0).start()

    @pl.when(i + 1 < ni)
    def _prefetch():
        block_copy(i + 1, 1 - slot).start()

    block_copy(i, slot).wait()

    lane = jax.lax.broadcasted_iota(jnp.int32, (R, VC), 1)
    row_v = (jax.lax.broadcasted_iota(jnp.int32, (R, VC), 0) + i * R) * V
    f_base = row_v + lane

    def chunk_val(j):
        lp = buf[slot, :, :, pl.ds(j * VC, VC)].reshape(R, VC)
        g = _gumbel_at(f_base + j * VC)
        return g + lp

    def body(j, carry):
        acc_v, acc_j = carry
        val = chunk_val(j)
        better = val > acc_v
        return (jnp.where(better, val, acc_v),
                jnp.where(better, jnp.int32(1) * (j * VC), acc_j))

    init = (chunk_val(0), jnp.zeros((R, VC), jnp.int32))
    acc_v, acc_j = jax.lax.fori_loop(1, NC, body, init, unroll=4)
    m1, f1 = _reduce_rowwise(acc_v, f_base + acc_j)

    if TW > 0:
        lane_t = jax.lax.broadcasted_iota(jnp.int32, (R, TW), 1)
        row_v_t = (jax.lax.broadcasted_iota(jnp.int32, (R, TW), 0) + i * R) * V
        f_t = row_v_t + lane_t + NC * VC
        lp_t = buf[slot, :, :, NC * VC:V].reshape(R, TW)
        val_t = _gumbel_at(f_t) + lp_t
        m2, f2 = _reduce_rowwise(val_t, f_t)
        tb = m2 > m1
        m1 = jnp.where(tb, m2, m1)
        f1 = jnp.where(tb, f2, f1)

    lp_win = m1 - _gumbel_at(f1)
    idx = f1 - (jax.lax.broadcasted_iota(jnp.int32, (R, 1), 0) + i * R) * V
    idx_ref[...] = idx.reshape(1, 1, R)
    score_ref[...] = lp_win.reshape(1, 1, R) + sc_ref[...]


def kernel(step, lprobs, scores):
    bsz, beam_size, V = lprobs.shape
    NROWS = bsz * beam_size
    BR = 4
    R = BR * beam_size
    VC = 512
    NC = V // VC
    ni = NROWS // R

    sc = scores.reshape(NROWS).reshape(ni, 1, R).astype(jnp.float32)

    idx3, score3 = pl.pallas_call(
        functools.partial(_sample_kernel, V, BR, beam_size, VC, NC),
        grid=(ni,),
        in_specs=[
            pl.BlockSpec(memory_space=pltpu.MemorySpace.HBM),
            pl.BlockSpec((1, 1, R), lambda i: (i, 0, 0)),
        ],
        out_specs=[
            pl.BlockSpec((1, 1, R), lambda i: (i, 0, 0)),
            pl.BlockSpec((1, 1, R), lambda i: (i, 0, 0)),
        ],
        out_shape=[
            jax.ShapeDtypeStruct((ni, 1, R), jnp.int32),
            jax.ShapeDtypeStruct((ni, 1, R), jnp.float32),
        ],
        scratch_shapes=[
            pltpu.VMEM((2, BR, beam_size, V), jnp.float32),
            pltpu.SemaphoreType.DMA((2,)),
        ],
        compiler_params=pltpu.CompilerParams(
            dimension_semantics=("arbitrary",),
        ),
    )(lprobs, sc)

    indices_buf = idx3.reshape(bsz, beam_size)
    scores_buf = score3.reshape(bsz, beam_size)
    beams_buf = jnp.tile(jnp.arange(beam_size, dtype=indices_buf.dtype), (bsz, 1))
    return (scores_buf, indices_buf, beams_buf)


# speedup vs baseline: 2.0707x; 1.2626x over previous
import functools

import jax
import jax.numpy as jnp
from jax.experimental import pallas as pl
from jax.experimental.pallas import tpu as pltpu

_TINY = 1.1754943508222875e-38
_BIG_I32 = 2**31 - 1
_KS = (0, 42, 0x1BD11BDA ^ 0 ^ 42)
_ROTS = ((13, 15, 26, 6), (17, 29, 16, 24))


def _rotl(x, r):
    return (x << jnp.uint32(r)) | (x >> jnp.uint32(32 - r))


def _threefry_bits(f_u32):
    x1 = f_u32 + jnp.uint32(_KS[1])
    x0 = x1
    x1 = _rotl(x1, _ROTS[0][0]) ^ x0
    for i in range(5):
        rots = _ROTS[i % 2]
        for r in (rots[1:] if i == 0 else rots):
            x0 = x0 + x1
            x1 = _rotl(x1, r) ^ x0
        a = _KS[(i + 1) % 3]
        b = (_KS[(i + 2) % 3] + i + 1) & 0xFFFFFFFF
        if a:
            x0 = x0 + jnp.uint32(a)
        x1 = x1 + jnp.uint32(b)
    return x0 ^ x1


def _gumbel_from_bits(bits):
    fb = (bits >> jnp.uint32(9)) | jnp.uint32(0x3F800000)
    u01 = jax.lax.bitcast_convert_type(fb, jnp.float32) - jnp.float32(1.0)
    tiny = jnp.float32(_TINY)
    u = jnp.maximum(tiny, u01 + tiny)
    return -jnp.log(-jnp.log(u))


def _gumbel_at(f_i32):
    return _gumbel_from_bits(
        _threefry_bits(jax.lax.bitcast_convert_type(f_i32, jnp.uint32)))


def _sample_kernel(BSZ, BEAM, V, VB, CH, NCI,
                   lp_ref, sc_ref, idx_ref, score_ref, av_scr, aj_scr):
    b = pl.program_id(0)
    j = pl.program_id(1)
    nj = pl.num_programs(1)

    sub = jax.lax.broadcasted_iota(jnp.int32, (CH, BSZ), 0)
    lane = jax.lax.broadcasted_iota(jnp.int32, (CH, BSZ), 1)
    f_lane = lane * (BEAM * V) + (b * V + j * VB)
    f_base = f_lane + sub

    @pl.when(j == 0)
    def _init():
        av_scr[...] = jnp.full((CH, BSZ), -jnp.inf, jnp.float32)
        aj_scr[...] = jnp.zeros((CH, BSZ), jnp.int32)

    def body(k, carry):
        acc_v, acc_j = carry
        lp = lp_ref[0, pl.ds(k * CH, CH), :]
        val = _gumbel_at(f_base + k * CH) + lp
        better = val > acc_v
        return (jnp.where(better, val, acc_v),
                jnp.where(better, jnp.int32(1) * (j * VB + k * CH), acc_j))

    acc = (av_scr[...], aj_scr[...])
    acc_v, acc_j = jax.lax.fori_loop(0, NCI, body, acc, unroll=5)
    av_scr[...] = acc_v
    aj_scr[...] = acc_j

    @pl.when(j == nj - 1)
    def _finalize():
        av = av_scr[...]
        aj = aj_scr[...]
        m = jnp.max(av, axis=0, keepdims=True)
        cand_v = aj + sub
        v_win = jnp.min(jnp.where(av == m, cand_v, jnp.int32(_BIG_I32)),
                        axis=0, keepdims=True)
        lane1 = jax.lax.broadcasted_iota(jnp.int32, (1, BSZ), 1)
        f_win = lane1 * (BEAM * V) + b * V + v_win
        lp_win = m - _gumbel_at(f_win)
        idx_ref[...] = v_win.reshape(1, 1, BSZ)
        score_ref[...] = (lp_win + sc_ref[0]).reshape(1, 1, BSZ)


def kernel(step, lprobs, scores):
    bsz, beam_size, V = lprobs.shape
    VB = 4000
    CH = 32
    NCI = VB // CH
    nb = V // VB

    lpT = jnp.transpose(lprobs, (1, 2, 0))
    scT = jnp.transpose(scores[:, :, 0], (1, 0)).reshape(beam_size, 1, bsz)

    idx3, score3 = pl.pallas_call(
        functools.partial(_sample_kernel, bsz, beam_size, V, VB, CH, NCI),
        grid=(beam_size, nb),
        in_specs=[
            pl.BlockSpec((1, VB, bsz), lambda b, j: (b, j, 0)),
            pl.BlockSpec((1, 1, bsz), lambda b, j: (b, 0, 0)),
        ],
        out_specs=[
            pl.BlockSpec((1, 1, bsz), lambda b, j: (b, 0, 0)),
            pl.BlockSpec((1, 1, bsz), lambda b, j: (b, 0, 0)),
        ],
        out_shape=[
            jax.ShapeDtypeStruct((beam_size, 1, bsz), jnp.int32),
            jax.ShapeDtypeStruct((beam_size, 1, bsz), jnp.float32),
        ],
        scratch_shapes=[
            pltpu.VMEM((CH, bsz), jnp.float32),
            pltpu.VMEM((CH, bsz), jnp.int32),
        ],
        compiler_params=pltpu.CompilerParams(
            dimension_semantics=("arbitrary", "arbitrary"),
        ),
    )(lpT, scT)

    indices_buf = jnp.transpose(idx3[:, 0, :], (1, 0))
    scores_buf = jnp.transpose(score3[:, 0, :], (1, 0))
    beams_buf = jnp.tile(jnp.arange(beam_size, dtype=indices_buf.dtype), (bsz, 1))
    return (scores_buf, indices_buf, beams_buf)
